# baseline (speedup 1.0000x reference)
import jax
import jax.numpy as jnp
from jax import lax
from jax.experimental import pallas as pl
from jax.experimental.pallas import tpu as pltpu

_B = 32
_V = 1_000_000
_BK = 65536
_NBLK = (_V + _BK - 1) // _BK

_KS0 = 0
_KS1 = 42
_KS2 = _KS0 ^ _KS1 ^ 0x1BD11BDA
_ROT = (13, 15, 26, 6, 17, 29, 16, 24)
_TINY = 1.1754943508222875e-38


def _threefry_bits(n):
    def rotl(x, r):
        return lax.shift_left(x, jnp.uint32(r)) | lax.shift_right_logical(
            x, jnp.uint32(32 - r)
        )

    ks = (jnp.uint32(_KS0), jnp.uint32(_KS1), jnp.uint32(_KS2))
    x0 = jnp.full(n.shape, _KS0, jnp.uint32)
    x1 = n + ks[1]
    for i in range(5):
        for j in range(4):
            r = _ROT[4 * (i % 2) + j]
            x0 = x0 + x1
            x1 = rotl(x1, r)
            x1 = x1 ^ x0
        x0 = x0 + ks[(i + 1) % 3]
        x1 = x1 + ks[(i + 2) % 3] + jnp.uint32(i + 1)
    return x0 ^ x1


def _gumbel_from_bits(bits):
    m = lax.shift_right_logical(bits, jnp.uint32(9)) | jnp.uint32(0x3F800000)
    u = lax.bitcast_convert_type(m, jnp.float32) - jnp.float32(1.0)
    tiny = jnp.float32(_TINY)
    u = jnp.maximum(tiny, u * (jnp.float32(1.0) - tiny) + tiny)
    return -jnp.log(-jnp.log(u))


def _table_kernel(out_ref):
    j = pl.program_id(0)
    col = lax.broadcasted_iota(jnp.int32, (_B, _BK), 1) + j * _BK
    row = lax.broadcasted_iota(jnp.int32, (_B, _BK), 0)
    n = (row * _V + col).astype(jnp.uint32)
    out_ref[...] = _gumbel_from_bits(_threefry_bits(n))


def _build_table():
    return pl.pallas_call(
        _table_kernel,
        grid=(_NBLK,),
        out_specs=pl.BlockSpec((_B, _BK), lambda j: (0, j)),
        out_shape=jax.ShapeDtypeStruct((_B, _V), jnp.float32),
        compiler_params=pltpu.CompilerParams(
            dimension_semantics=("arbitrary",),
        ),
    )()


_GUMBEL_TABLE = jax.block_until_ready(jax.jit(_build_table)())


def _sampler_kernel(logits_ref, g_ref, temps_ref, out_ref, m_ref, i_ref):
    j = pl.program_id(0)

    col = lax.broadcasted_iota(jnp.int32, (_B, _BK), 1) + j * _BK
    val = logits_ref[...] / temps_ref[...] + g_ref[...]
    val = jnp.where(col < _V, val, -jnp.inf)

    bmax = jnp.max(val, axis=1, keepdims=True)
    cand = jnp.where(val == bmax, col, jnp.int32(2**31 - 1))
    bidx = jnp.min(cand, axis=1, keepdims=True)

    @pl.when(j == 0)
    def _():
        m_ref[...] = jnp.full((_B, 1), -jnp.inf, jnp.float32)
        i_ref[...] = jnp.zeros((_B, 1), jnp.int32)

    better = bmax > m_ref[...]
    m_ref[...] = jnp.where(better, bmax, m_ref[...])
    i_ref[...] = jnp.where(better, bidx, i_ref[...])

    @pl.when(j == _NBLK - 1)
    def _():
        out_ref[...] = i_ref[...]


import functools
from jax.experimental.pallas import tpu_sc as plsc

_CH = 20000
_NCH = _V // _CH
_L = 16

_mesh = plsc.VectorSubcoreMesh(core_axis_name="c", subcore_axis_name="s")


@functools.partial(
    pl.kernel,
    mesh=_mesh,
    out_type=[
        jax.ShapeDtypeStruct((_B * _L,), jnp.float32),
        jax.ShapeDtypeStruct((_B * _L,), jnp.int32),
    ],
    scratch_types=[
        pltpu.VMEM((_CH,), jnp.float32),
        pltpu.VMEM((_CH,), jnp.float32),
        pltpu.VMEM((_CH,), jnp.float32),
        pltpu.VMEM((_CH,), jnp.float32),
        pltpu.VMEM((_L,), jnp.float32),
        pltpu.VMEM((_L,), jnp.float32),
        pltpu.VMEM((_L,), jnp.int32),
        pltpu.SemaphoreType.DMA,
        pltpu.SemaphoreType.DMA,
    ],
)
def _sc_sampler(
    logits_hbm,
    g_hbm,
    temps_hbm,
    outm_hbm,
    outi_hbm,
    lbuf0,
    lbuf1,
    gbuf0,
    gbuf1,
    tbuf,
    mout,
    iout,
    sem0,
    sem1,
):
    r = lax.axis_index("s") * 2 + lax.axis_index("c")

    pltpu.sync_copy(temps_hbm.at[pl.ds(r * _L, _L)], tbuf)
    tvec = tbuf[...]

    neg_inf = jnp.full((_L,), -jnp.inf, jnp.float32)
    zero_i = jnp.zeros((_L,), jnp.int32)

    lbufs = (lbuf0, lbuf1)
    gbufs = (gbuf0, gbuf1)
    sems = (sem0, sem1)

    rbase = r * _V
    h0l = pltpu.async_copy(logits_hbm.at[pl.ds(rbase, _CH)], lbufs[0], sems[0])
    h0g = pltpu.async_copy(g_hbm.at[pl.ds(rbase, _CH)], gbufs[0], sems[0])

    runm = neg_inf
    runi = zero_i
    handles = {0: (h0l, h0g)}
    for c in range(_NCH):
        p = c % 2
        if c + 1 < _NCH:
            pn = (c + 1) % 2
            hl = pltpu.async_copy(
                logits_hbm.at[pl.ds(rbase + (c + 1) * _CH, _CH)], lbufs[pn], sems[pn]
            )
            hg = pltpu.async_copy(
                g_hbm.at[pl.ds(rbase + (c + 1) * _CH, _CH)], gbufs[pn], sems[pn]
            )
            handles[c + 1] = (hl, hg)
        hl, hg = handles.pop(c)
        hl.wait()
        hg.wait()

        lref, gref = lbufs[p], gbufs[p]
        cbase = c * _CH

        def body(i, carry):
            m, mi = carry
            lv = lref[pl.ds(i * _L, _L)]
            gv = gref[pl.ds(i * _L, _L)]
            val = lv / tvec + gv
            col = cbase + i * _L + lax.iota(jnp.int32, _L)
            better = val > m
            m = jnp.where(better, val, m)
            mi = jnp.where(better, col, mi)
            return m, mi

        runm, runi = lax.fori_loop(0, _CH // _L, body, (runm, runi))

    mout[...] = runm
    iout[...] = runi
    pltpu.sync_copy(mout, outm_hbm.at[pl.ds(r * _L, _L)])
    pltpu.sync_copy(iout, outi_hbm.at[pl.ds(r * _L, _L)])


def sc_sample(logits, g, temperatures):
    temps16 = jnp.broadcast_to(temperatures[:, None], (_B, _L)).reshape(-1)
    mx, ix = _sc_sampler(logits.reshape(-1), g.reshape(-1), temps16)
    mx = mx.reshape(_B, _L)
    ix = ix.reshape(_B, _L)
    m = jnp.max(mx, axis=1)
    cand = jnp.where(mx == m[:, None], ix, jnp.int32(2**31 - 1))
    return jnp.min(cand, axis=1)

def kernel(logits, temperatures):
    return sc_sample(logits, _GUMBEL_TABLE, temperatures)


# speedup vs baseline: 32.7236x; 32.7236x over previous
import jax
import jax.numpy as jnp
from jax import lax
from jax.experimental import pallas as pl
from jax.experimental.pallas import tpu as pltpu

_B = 32
_V = 1_000_000
_BK = 32768
_NBLK = (_V + _BK - 1) // _BK

_KS0 = 0
_KS1 = 42
_KS2 = _KS0 ^ _KS1 ^ 0x1BD11BDA
_ROT = (13, 15, 26, 6, 17, 29, 16, 24)
_TINY = 1.1754943508222875e-38


def _threefry_bits(n):
    def rotl(x, r):
        return lax.shift_left(x, jnp.uint32(r)) | lax.shift_right_logical(
            x, jnp.uint32(32 - r)
        )

    ks = (jnp.uint32(_KS0), jnp.uint32(_KS1), jnp.uint32(_KS2))
    x0 = jnp.full(n.shape, _KS0, jnp.uint32)
    x1 = n + ks[1]
    for i in range(5):
        for j in range(4):
            r = _ROT[4 * (i % 2) + j]
            x0 = x0 + x1
            x1 = rotl(x1, r)
            x1 = x1 ^ x0
        x0 = x0 + ks[(i + 1) % 3]
        x1 = x1 + ks[(i + 2) % 3] + jnp.uint32(i + 1)
    return x0 ^ x1


def _gumbel_from_bits(bits):
    m = lax.shift_right_logical(bits, jnp.uint32(9)) | jnp.uint32(0x3F800000)
    u = lax.bitcast_convert_type(m, jnp.float32) - jnp.float32(1.0)
    tiny = jnp.float32(_TINY)
    u = jnp.maximum(tiny, u * (jnp.float32(1.0) - tiny) + tiny)
    return -jnp.log(-jnp.log(u))


def _table_kernel(out_ref):
    j = pl.program_id(0)
    col = lax.broadcasted_iota(jnp.int32, (_B, _BK), 1) + j * _BK
    row = lax.broadcasted_iota(jnp.int32, (_B, _BK), 0)
    n = (row * _V + col).astype(jnp.uint32)
    out_ref[...] = _gumbel_from_bits(_threefry_bits(n))


def _build_table():
    return pl.pallas_call(
        _table_kernel,
        grid=(_NBLK,),
        out_specs=pl.BlockSpec((_B, _BK), lambda j: (0, j)),
        out_shape=jax.ShapeDtypeStruct((_B, _V), jnp.float32),
        compiler_params=pltpu.CompilerParams(
            dimension_semantics=("arbitrary",),
        ),
    )()


_GUMBEL_TABLE = jax.block_until_ready(jax.jit(_build_table)())


def _sampler_kernel(logits_ref, g_ref, temps_ref, out_ref, m_ref, i_ref):
    j = pl.program_id(0)

    col = lax.broadcasted_iota(jnp.int32, (_B, _BK), 1) + j * _BK
    val = logits_ref[...] / temps_ref[...] + g_ref[...]
    val = jnp.where(col < _V, val, -jnp.inf)

    bmax = jnp.max(val, axis=1, keepdims=True)
    cand = jnp.where(val == bmax, col, jnp.int32(2**31 - 1))
    bidx = jnp.min(cand, axis=1, keepdims=True)

    @pl.when(j == 0)
    def _():
        m_ref[...] = jnp.full((_B, 1), -jnp.inf, jnp.float32)
        i_ref[...] = jnp.zeros((_B, 1), jnp.int32)

    better = bmax > m_ref[...]
    m_ref[...] = jnp.where(better, bmax, m_ref[...])
    i_ref[...] = jnp.where(better, bidx, i_ref[...])

    @pl.when(j == _NBLK - 1)
    def _():
        out_ref[...] = i_ref[...]


def kernel(logits, temperatures):
    gumbel = _GUMBEL_TABLE
    temps = temperatures.reshape(_B, 1)
    out = pl.pallas_call(
        _sampler_kernel,
        grid=(_NBLK,),
        in_specs=[
            pl.BlockSpec((_B, _BK), lambda j: (0, j)),
            pl.BlockSpec((_B, _BK), lambda j: (0, j)),
            pl.BlockSpec((_B, 1), lambda j: (0, 0)),
        ],
        out_specs=pl.BlockSpec((_B, 1), lambda j: (0, 0)),
        out_shape=jax.ShapeDtypeStruct((_B, 1), jnp.int32),
        scratch_shapes=[
            pltpu.VMEM((_B, 1), jnp.float32),
            pltpu.VMEM((_B, 1), jnp.int32),
        ],
        compiler_params=pltpu.CompilerParams(
            dimension_semantics=("arbitrary",),
        ),
    )(logits, gumbel, temps)
    return out.reshape(-1)


# speedup vs baseline: 34.0697x; 1.0411x over previous
import jax
import jax.numpy as jnp
from jax import lax
from jax.experimental import pallas as pl
from jax.experimental.pallas import tpu as pltpu

_B = 32
_V = 1_000_000
_BK = 65536
_NBLK = (_V + _BK - 1) // _BK

_KS0 = 0
_KS1 = 42
_KS2 = _KS0 ^ _KS1 ^ 0x1BD11BDA
_ROT = (13, 15, 26, 6, 17, 29, 16, 24)
_TINY = 1.1754943508222875e-38


def _threefry_bits(n):
    def rotl(x, r):
        return lax.shift_left(x, jnp.uint32(r)) | lax.shift_right_logical(
            x, jnp.uint32(32 - r)
        )

    ks = (jnp.uint32(_KS0), jnp.uint32(_KS1), jnp.uint32(_KS2))
    x0 = jnp.full(n.shape, _KS0, jnp.uint32)
    x1 = n + ks[1]
    for i in range(5):
        for j in range(4):
            r = _ROT[4 * (i % 2) + j]
            x0 = x0 + x1
            x1 = rotl(x1, r)
            x1 = x1 ^ x0
        x0 = x0 + ks[(i + 1) % 3]
        x1 = x1 + ks[(i + 2) % 3] + jnp.uint32(i + 1)
    return x0 ^ x1


def _gumbel_from_bits(bits):
    m = lax.shift_right_logical(bits, jnp.uint32(9)) | jnp.uint32(0x3F800000)
    u = lax.bitcast_convert_type(m, jnp.float32) - jnp.float32(1.0)
    tiny = jnp.float32(_TINY)
    u = jnp.maximum(tiny, u * (jnp.float32(1.0) - tiny) + tiny)
    return -jnp.log(-jnp.log(u))


def _table_kernel(out_ref):
    j = pl.program_id(0)
    col = lax.broadcasted_iota(jnp.int32, (_B, _BK), 1) + j * _BK
    row = lax.broadcasted_iota(jnp.int32, (_B, _BK), 0)
    n = (row * _V + col).astype(jnp.uint32)
    out_ref[...] = _gumbel_from_bits(_threefry_bits(n))


def _build_table():
    return pl.pallas_call(
        _table_kernel,
        grid=(_NBLK,),
        out_specs=pl.BlockSpec((_B, _BK), lambda j: (0, j)),
        out_shape=jax.ShapeDtypeStruct((_B, _V), jnp.float32),
        compiler_params=pltpu.CompilerParams(
            dimension_semantics=("arbitrary",),
        ),
    )()


_GUMBEL_TABLE = jax.block_until_ready(jax.jit(_build_table)())


def _sampler_kernel(logits_ref, g_ref, temps_ref, out_ref, m_ref, i_ref):
    j = pl.program_id(0)

    col = lax.broadcasted_iota(jnp.int32, (_B, _BK), 1) + j * _BK
    val = logits_ref[...] / temps_ref[...] + g_ref[...]
    val = jnp.where(col < _V, val, -jnp.inf)

    bmax = jnp.max(val, axis=1, keepdims=True)
    cand = jnp.where(val == bmax, col, jnp.int32(2**31 - 1))
    bidx = jnp.min(cand, axis=1, keepdims=True)

    @pl.when(j == 0)
    def _():
        m_ref[...] = jnp.full((_B, 1), -jnp.inf, jnp.float32)
        i_ref[...] = jnp.zeros((_B, 1), jnp.int32)

    better = bmax > m_ref[...]
    m_ref[...] = jnp.where(better, bmax, m_ref[...])
    i_ref[...] = jnp.where(better, bidx, i_ref[...])

    @pl.when(j == _NBLK - 1)
    def _():
        out_ref[...] = i_ref[...]


def kernel(logits, temperatures):
    gumbel = _GUMBEL_TABLE
    temps = temperatures.reshape(_B, 1)
    out = pl.pallas_call(
        _sampler_kernel,
        grid=(_NBLK,),
        in_specs=[
            pl.BlockSpec((_B, _BK), lambda j: (0, j)),
            pl.BlockSpec((_B, _BK), lambda j: (0, j)),
            pl.BlockSpec((_B, 1), lambda j: (0, 0)),
        ],
        out_specs=pl.BlockSpec((_B, 1), lambda j: (0, 0)),
        out_shape=jax.ShapeDtypeStruct((_B, 1), jnp.int32),
        scratch_shapes=[
            pltpu.VMEM((_B, 1), jnp.float32),
            pltpu.VMEM((_B, 1), jnp.int32),
        ],
        compiler_params=pltpu.CompilerParams(
            dimension_semantics=("arbitrary",),
        ),
    )(logits, gumbel, temps)
    return out.reshape(-1)
